# Initial kernel scaffold; baseline (speedup 1.0000x reference)
import jax
import jax.numpy as jnp
from jax.experimental import pallas as pl


def kernel(nodes, x, idx_labeled, labels, neigh, W1, W2, weight):
    raise NotImplementedError("write your pallas kernel here")


# speedup vs baseline: 6.6682x; 6.6682x over previous
import functools

import jax
import jax.numpy as jnp
from jax import lax
from jax.experimental import pallas as pl
from jax.experimental.pallas import tpu as pltpu
from jax.experimental.pallas import tpu_sc as plsc

NC = 2
NSUB = 16
NW = NC * NSUB
L = 16


def _sc_gather_kernel(n, d, bsz, ns, nlab, nclass, h1d, h1p, nrows):
    bt = bsz // NW
    half = bt // 2
    npad = ((n + L - 1) // L) * L
    padl = (-nlab) % L
    npadl = nlab + padl
    ngrp = ns + 1
    npg = ns + 1

    mesh = plsc.VectorSubcoreMesh(core_axis_name="c", subcore_axis_name="s")

    def body(nodes_h, x_h, il_h, lab_h, nf_h, wc_h,
             xs_h, cs_h,
             il_v, lv_v, cls_v, nod_v, rix_v, rbuf, nb1c_v, cid_v,
             b0, b1, b2, b3, csum_v, wc_v, sem):
        iota = lax.iota(jnp.int32, L)
        czid = lax.axis_index("c")
        sid = lax.axis_index("s")
        wid = sid * NC + czid
        base = pl.multiple_of(wid * bt, bt)

        pltpu.sync_copy(wc_h, wc_v)

        pltpu.sync_copy(lab_h, cls_v.at[pl.ds(0, n)])
        pltpu.sync_copy(il_h, il_v.at[pl.ds(0, nlab)])
        if padl:
            pltpu.sync_copy(il_h.at[pl.ds(0, padl)],
                            il_v.at[pl.ds(nlab, padl)])

        def _gl(i, c):
            sl = pl.ds(pl.multiple_of(i * L, L), L)
            lv_v[sl] = plsc.load_gather(cls_v, [il_v[sl]])
            return c
        lax.fori_loop(0, npadl // L, _gl, 0)

        unl = jnp.full((L,), nclass, jnp.int32)

        def _ms(i, c):
            cls_v[pl.ds(pl.multiple_of(i * L, L), L)] = unl
            return c
        lax.fori_loop(0, npad // L, _ms, 0)

        def _sc(i, c):
            sl = pl.ds(pl.multiple_of(i * L, L), L)
            plsc.store_scatter(cls_v, [il_v[sl]], lv_v[sl])
            return c
        lax.fori_loop(0, npadl // L, _sc, 0)

        pltpu.sync_copy(nodes_h.at[pl.ds(base, bt)], nod_v)

        zf = jnp.zeros((L,), jnp.float32)

        def _zc(r, c):
            for q in range(h1p // L):
                csum_v[r, pl.ds(q * L, L)] = zf
            return c
        lax.fori_loop(0, bt, _zc, 0)

        def _ri0(j, c):
            sl = pl.ds(pl.multiple_of(j * L, L), L)
            rix_v[sl] = lax.shift_right_logical(nod_v[sl], 5)
            return c
        lax.fori_loop(0, bt // L, _ri0, 0)
        pltpu.async_copy(nf_h.at[rix_v], rbuf, sem).wait()

        def _ex0(j, c):
            sl = pl.ds(pl.multiple_of(j * L, L), L)
            rows = j * L + iota
            idv = nod_v[sl]
            cbase = (idv & 31) * npg
            for k in range(ns):
                nb1c_v[k, sl] = plsc.load_gather(rbuf, [rows, cbase + k])
            return c
        lax.fori_loop(0, bt // L, _ex0, 0)

        for g in range(ngrp):
            def _cp(j, c):
                sl = pl.ds(pl.multiple_of(j * L, L), L)
                if g == 0:
                    cid_v[0, sl] = nod_v[sl]
                else:
                    cid_v[0, sl] = nb1c_v[g - 1, sl]
                return c
            lax.fori_loop(0, bt // L, _cp, 0)

            if g == 0:
                def _cc(j, c):
                    sl = pl.ds(pl.multiple_of(j * L, L), L)
                    for k in range(ns):
                        cid_v[k + 1, sl] = nb1c_v[k, sl]
                    return c
                lax.fori_loop(0, bt // L, _cc, 0)
            else:
                def _ri(j, c):
                    sl = pl.ds(pl.multiple_of(j * L, L), L)
                    rix_v[sl] = lax.shift_right_logical(cid_v[0, sl], 5)
                    return c
                lax.fori_loop(0, bt // L, _ri, 0)
                pltpu.async_copy(nf_h.at[rix_v], rbuf, sem).wait()

                def _ex(j, c):
                    sl = pl.ds(pl.multiple_of(j * L, L), L)
                    rows = j * L + iota
                    idv = cid_v[0, sl]
                    cbase = (idv & 31) * npg
                    for k in range(ns):
                        cid_v[k + 1, sl] = plsc.load_gather(
                            rbuf, [rows, cbase + k])
                    return c
                lax.fori_loop(0, bt // L, _ex, 0)

            row0 = pl.multiple_of(g * bsz + base, bt)
            for hh in range(2):
                offs = hh * half
                bufs = (b0, b1, b2, b3)
                dlist = [
                    pltpu.async_copy(
                        x_h.at[cid_v.at[k, pl.ds(offs, half)]], bufs[k], sem)
                    for k in range(ngrp)
                ]

                def _cs(j, c):
                    jj = hh * (half // L) + j
                    sl = pl.ds(pl.multiple_of(jj * L, L), L)
                    rows = jj * L + iota
                    cks = [plsc.load_gather(cls_v, [cid_v[k, sl]])
                           for k in range(ngrp)]
                    for dcol in range(h1d):
                        dv = jnp.full((L,), dcol, jnp.int32)
                        s = plsc.load_gather(wc_v, [cks[0], dv])
                        for ck in cks[1:]:
                            s = s + plsc.load_gather(wc_v, [ck, dv])
                        plsc.store_scatter(csum_v, [rows, dv], s)
                    return c
                lax.fori_loop(0, half // L, _cs, 0)
                for dsc in dlist:
                    dsc.wait()

                def _acc(r, c):
                    for q in range(d // L):
                        sl = pl.ds(q * L, L)
                        b0[r, sl] = (b0[r, sl] + b1[r, sl]
                                     + b2[r, sl] + b3[r, sl])
                    return c
                lax.fori_loop(0, half, _acc, 0)
                pltpu.sync_copy(
                    b0, xs_h.at[pl.ds(pl.multiple_of(row0 + offs, half),
                                      half)])
            pltpu.sync_copy(csum_v, cs_h.at[pl.ds(row0, bt)])

    return pl.kernel(
        body,
        out_type=[
            jax.ShapeDtypeStruct((ngrp * bsz, d), jnp.float32),
            jax.ShapeDtypeStruct((ngrp * bsz, h1p), jnp.float32),
        ],
        mesh=mesh,
        compiler_params=pltpu.CompilerParams(needs_layout_passes=False),
        scratch_types=[
            pltpu.VMEM((npadl,), jnp.int32),
            pltpu.VMEM((npadl,), jnp.int32),
            pltpu.VMEM((npad,), jnp.int32),
            pltpu.VMEM((bt,), jnp.int32),
            pltpu.VMEM((bt,), jnp.int32),
            pltpu.VMEM((bt, 128), jnp.int32),
            pltpu.VMEM((ns, bt), jnp.int32),
            pltpu.VMEM((ns + 1, bt), jnp.int32),
            pltpu.VMEM((bt // 2, d), jnp.float32),
            pltpu.VMEM((bt // 2, d), jnp.float32),
            pltpu.VMEM((bt // 2, d), jnp.float32),
            pltpu.VMEM((bt // 2, d), jnp.float32),
            pltpu.VMEM((bt, h1p), jnp.float32),
            pltpu.VMEM((nclass + 1, h1p), jnp.float32),
            pltpu.SemaphoreType.DMA,
        ],
    )


def _tc_body(xs_ref, cs_ref, w1a_ref, w2t_ref, wt_ref, o_ref, *, ngrp, rb, d):
    x4 = xs_ref[...]
    pre = jnp.dot(x4.reshape(ngrp * rb, d), w1a_ref[...],
                  preferred_element_type=jnp.float32)
    h1p = cs_ref.shape[-1]
    pre = (pre.reshape(ngrp, rb, h1p) + cs_ref[...]) * (1.0 / ngrp)
    h1 = jnp.maximum(pre, 0.0)
    agg2 = jnp.sum(h1, axis=0) * (1.0 / ngrp)
    h2 = jnp.maximum(
        jnp.dot(agg2, w2t_ref[...], preferred_element_type=jnp.float32), 0.0)
    sc = jnp.dot(h2, wt_ref[...], preferred_element_type=jnp.float32)
    m = jnp.max(sc, axis=1, keepdims=True)
    sh = sc - m
    o_ref[...] = sh - jnp.log(jnp.sum(jnp.exp(sh), axis=1, keepdims=True))


@jax.jit
def kernel(nodes, x, idx_labeled, labels, neigh, W1, W2, weight):
    n, d = x.shape
    bsz = nodes.shape[0]
    ns = neigh.shape[1]
    nlab = idx_labeled.shape[0]
    nclass = weight.shape[0]
    h1 = W1.shape[0]
    h2 = W2.shape[0]
    ngrp = ns + 1
    h1p = ((h1 + L - 1) // L) * L
    h2p = ((h2 + 15) // 16) * 16

    npg = ns + 1
    flat = jnp.pad(neigh, ((0, 0), (0, npg - ns))).reshape(-1)
    nrows = (n * npg + 127) // 128
    flat = jnp.pad(flat, (0, nrows * 128 - n * npg)).reshape(nrows, 128)

    wc = jnp.zeros((nclass + 1, h1p), jnp.float32)
    wc = wc.at[:nclass, :h1].set(W1[:, d:].T)
    w1a = jnp.zeros((d, h1p), jnp.float32).at[:, :h1].set(W1[:, :d].T)
    w2t = jnp.zeros((h1p, h2p), jnp.float32).at[:h1, :h2].set(W2.T)
    wt = jnp.zeros((h2p, nclass), jnp.float32).at[:h2, :].set(weight.T)

    xs, csum = _sc_gather_kernel(n, d, bsz, ns, nlab, nclass, h1, h1p,
                                 nrows)(
        nodes, x, idx_labeled, labels, flat, wc)

    xs4 = xs.reshape(ngrp, bsz, d)
    cs4 = csum.reshape(ngrp, bsz, h1p)

    rb = 1024
    grid = (bsz // rb,)
    out = pl.pallas_call(
        functools.partial(_tc_body, ngrp=ngrp, rb=rb, d=d),
        grid=grid,
        in_specs=[
            pl.BlockSpec((ngrp, rb, d), lambda i: (0, i, 0)),
            pl.BlockSpec((ngrp, rb, h1p), lambda i: (0, i, 0)),
            pl.BlockSpec((d, h1p), lambda i: (0, 0)),
            pl.BlockSpec((h1p, h2p), lambda i: (0, 0)),
            pl.BlockSpec((h2p, nclass), lambda i: (0, 0)),
        ],
        out_specs=pl.BlockSpec((rb, nclass), lambda i: (i, 0)),
        out_shape=jax.ShapeDtypeStruct((bsz, nclass), jnp.float32),
    )(xs4, cs4, w1a, w2t, wt)
    return out


# speedup vs baseline: 8.3733x; 1.2557x over previous
import functools

import jax
import jax.numpy as jnp
from jax import lax
from jax.experimental import pallas as pl
from jax.experimental.pallas import tpu as pltpu
from jax.experimental.pallas import tpu_sc as plsc

NC = 2
NSUB = 16
NW = NC * NSUB
L = 16


def _sc_gather_kernel(n, d, bsz, ns, nlab, nclass, h1d, h1p, nrows):
    bt = bsz // NW
    half = bt // 2
    npad = ((n + L - 1) // L) * L
    padl = (-nlab) % L
    npadl = nlab + padl
    ngrp = ns + 1
    npg = ns + 1

    mesh = plsc.VectorSubcoreMesh(core_axis_name="c", subcore_axis_name="s")

    def body(nodes_h, x_h, il_h, lab_h, nf_h, wc_h,
             xs_h, cs_h,
             il_v, lv_v, cls_v, nod_v, rix_v, rbuf, nb1c_v, cid_v,
             b0, b1, b2, b3, csum_v, wc_v, sem):
        iota = lax.iota(jnp.int32, L)
        czid = lax.axis_index("c")
        sid = lax.axis_index("s")
        wid = sid * NC + czid
        base = pl.multiple_of(wid * bt, bt)

        pltpu.sync_copy(wc_h, wc_v)

        pltpu.sync_copy(lab_h, cls_v.at[pl.ds(0, n)])
        pltpu.sync_copy(il_h, il_v.at[pl.ds(0, nlab)])
        if padl:
            pltpu.sync_copy(il_h.at[pl.ds(0, padl)],
                            il_v.at[pl.ds(nlab, padl)])

        def _gl(i, c):
            sl = pl.ds(pl.multiple_of(i * L, L), L)
            lv_v[sl] = plsc.load_gather(cls_v, [il_v[sl]])
            return c
        lax.fori_loop(0, npadl // L, _gl, 0)

        unl = jnp.full((L,), nclass, jnp.int32)

        def _ms(i, c):
            for u in range(8):
                cls_v[pl.ds(pl.multiple_of(i * 8 * L + u * L, L), L)] = unl
            return c
        lax.fori_loop(0, npad // (8 * L), _ms, 0)

        def _sc(i, c):
            sl = pl.ds(pl.multiple_of(i * L, L), L)
            plsc.store_scatter(cls_v, [il_v[sl]], lv_v[sl])
            return c
        lax.fori_loop(0, npadl // L, _sc, 0)

        pltpu.sync_copy(nodes_h.at[pl.ds(base, bt)], nod_v)

        zf = jnp.zeros((L,), jnp.float32)

        def _zc(r, c):
            for q in range(h1p // L):
                csum_v[r, pl.ds(q * L, L)] = zf
            return c
        lax.fori_loop(0, bt, _zc, 0)

        pltpu.async_copy(nf_h.at[nod_v], rbuf, sem).wait()

        def _ex0(j, c):
            sl = pl.ds(pl.multiple_of(j * L, L), L)
            rows = j * L + iota
            for k in range(ns):
                ck = jnp.full((L,), k, jnp.int32)
                nb1c_v[k, sl] = plsc.load_gather(rbuf, [rows, ck])
            return c
        lax.fori_loop(0, bt // L, _ex0, 0)

        for g in range(ngrp):
            def _cp(j, c):
                sl = pl.ds(pl.multiple_of(j * L, L), L)
                if g == 0:
                    cid_v[0, sl] = nod_v[sl]
                else:
                    cid_v[0, sl] = nb1c_v[g - 1, sl]
                return c
            lax.fori_loop(0, bt // L, _cp, 0)

            if g == 0:
                def _cc(j, c):
                    sl = pl.ds(pl.multiple_of(j * L, L), L)
                    for k in range(ns):
                        cid_v[k + 1, sl] = nb1c_v[k, sl]
                    return c
                lax.fori_loop(0, bt // L, _cc, 0)
            else:
                pltpu.async_copy(nf_h.at[cid_v.at[0]], rbuf, sem).wait()

                def _ex(j, c):
                    sl = pl.ds(pl.multiple_of(j * L, L), L)
                    rows = j * L + iota
                    for k in range(ns):
                        ck = jnp.full((L,), k, jnp.int32)
                        cid_v[k + 1, sl] = plsc.load_gather(
                            rbuf, [rows, ck])
                    return c
                lax.fori_loop(0, bt // L, _ex, 0)

            row0 = pl.multiple_of(g * bsz + base, bt)
            for hh in range(2):
                offs = hh * half
                bufs = (b0, b1, b2, b3)
                dlist = [
                    pltpu.async_copy(
                        x_h.at[cid_v.at[k, pl.ds(offs, half)]], bufs[k], sem)
                    for k in range(ngrp)
                ]

                def _cs(j, c):
                    jj = hh * (half // L) + j
                    sl = pl.ds(pl.multiple_of(jj * L, L), L)
                    rows = jj * L + iota
                    cks = [plsc.load_gather(cls_v, [cid_v[k, sl]])
                           for k in range(ngrp)]
                    for dcol in range(h1d):
                        dv = jnp.full((L,), dcol, jnp.int32)
                        s = plsc.load_gather(wc_v, [cks[0], dv])
                        for ck in cks[1:]:
                            s = s + plsc.load_gather(wc_v, [ck, dv])
                        plsc.store_scatter(csum_v, [rows, dv], s)
                    return c
                lax.fori_loop(0, half // L, _cs, 0)
                for dsc in dlist:
                    dsc.wait()

                def _acc(r, c):
                    for q in range(d // L):
                        sl = pl.ds(q * L, L)
                        b0[r, sl] = (b0[r, sl] + b1[r, sl]
                                     + b2[r, sl] + b3[r, sl])
                    return c
                lax.fori_loop(0, half, _acc, 0)
                pltpu.sync_copy(
                    b0, xs_h.at[pl.ds(pl.multiple_of(row0 + offs, half),
                                      half)])
            pltpu.sync_copy(csum_v, cs_h.at[pl.ds(row0, bt)])

    return pl.kernel(
        body,
        out_type=[
            jax.ShapeDtypeStruct((ngrp * bsz, d), jnp.float32),
            jax.ShapeDtypeStruct((ngrp * bsz, h1p), jnp.float32),
        ],
        mesh=mesh,
        compiler_params=pltpu.CompilerParams(needs_layout_passes=False),
        scratch_types=[
            pltpu.VMEM((npadl,), jnp.int32),
            pltpu.VMEM((npadl,), jnp.int32),
            pltpu.VMEM((npad,), jnp.int32),
            pltpu.VMEM((bt,), jnp.int32),
            pltpu.VMEM((bt,), jnp.int32),
            pltpu.VMEM((bt, 128), jnp.int32),
            pltpu.VMEM((ns, bt), jnp.int32),
            pltpu.VMEM((ns + 1, bt), jnp.int32),
            pltpu.VMEM((bt // 2, d), jnp.float32),
            pltpu.VMEM((bt // 2, d), jnp.float32),
            pltpu.VMEM((bt // 2, d), jnp.float32),
            pltpu.VMEM((bt // 2, d), jnp.float32),
            pltpu.VMEM((bt, h1p), jnp.float32),
            pltpu.VMEM((nclass + 1, h1p), jnp.float32),
            pltpu.SemaphoreType.DMA,
        ],
    )


def _tc_body(xs_ref, cs_ref, w1a_ref, w2t_ref, wt_ref, o_ref, *, ngrp, rb, d):
    x4 = xs_ref[...]
    pre = jnp.dot(x4.reshape(ngrp * rb, d), w1a_ref[...],
                  preferred_element_type=jnp.float32)
    h1p = cs_ref.shape[-1]
    pre = (pre.reshape(ngrp, rb, h1p) + cs_ref[...]) * (1.0 / ngrp)
    h1 = jnp.maximum(pre, 0.0)
    agg2 = jnp.sum(h1, axis=0) * (1.0 / ngrp)
    h2 = jnp.maximum(
        jnp.dot(agg2, w2t_ref[...], preferred_element_type=jnp.float32), 0.0)
    sc = jnp.dot(h2, wt_ref[...], preferred_element_type=jnp.float32)
    m = jnp.max(sc, axis=1, keepdims=True)
    sh = sc - m
    o_ref[...] = sh - jnp.log(jnp.sum(jnp.exp(sh), axis=1, keepdims=True))


@jax.jit
def kernel(nodes, x, idx_labeled, labels, neigh, W1, W2, weight):
    n, d = x.shape
    bsz = nodes.shape[0]
    ns = neigh.shape[1]
    nlab = idx_labeled.shape[0]
    nclass = weight.shape[0]
    h1 = W1.shape[0]
    h2 = W2.shape[0]
    ngrp = ns + 1
    h1p = ((h1 + L - 1) // L) * L
    h2p = ((h2 + 15) // 16) * 16

    flat = jnp.pad(neigh, ((0, 0), (0, 128 - ns)))
    nrows = n

    wc = jnp.zeros((nclass + 1, h1p), jnp.float32)
    wc = wc.at[:nclass, :h1].set(W1[:, d:].T)
    w1a = jnp.zeros((d, h1p), jnp.float32).at[:, :h1].set(W1[:, :d].T)
    w2t = jnp.zeros((h1p, h2p), jnp.float32).at[:h1, :h2].set(W2.T)
    wt = jnp.zeros((h2p, nclass), jnp.float32).at[:h2, :].set(weight.T)

    xs, csum = _sc_gather_kernel(n, d, bsz, ns, nlab, nclass, h1, h1p,
                                 nrows)(
        nodes, x, idx_labeled, labels, flat, wc)

    xs4 = xs.reshape(ngrp, bsz, d)
    cs4 = csum.reshape(ngrp, bsz, h1p)

    rb = 1024
    grid = (bsz // rb,)
    out = pl.pallas_call(
        functools.partial(_tc_body, ngrp=ngrp, rb=rb, d=d),
        grid=grid,
        in_specs=[
            pl.BlockSpec((ngrp, rb, d), lambda i: (0, i, 0)),
            pl.BlockSpec((ngrp, rb, h1p), lambda i: (0, i, 0)),
            pl.BlockSpec((d, h1p), lambda i: (0, 0)),
            pl.BlockSpec((h1p, h2p), lambda i: (0, 0)),
            pl.BlockSpec((h2p, nclass), lambda i: (0, 0)),
        ],
        out_specs=pl.BlockSpec((rb, nclass), lambda i: (i, 0)),
        out_shape=jax.ShapeDtypeStruct((bsz, nclass), jnp.float32),
    )(xs4, cs4, w1a, w2t, wt)
    return out
